# baseline (speedup 1.0000x reference)
import functools

import jax
import jax.numpy as jnp
from jax import lax
from jax.experimental import pallas as pl
from jax.experimental.pallas import tpu as pltpu
from jax.experimental.pallas import tpu_sc as plsc

D_VOCAB = 100000
D_MODEL = 1024
BATCH = 4
POS = 2048

NC = 2
NS = 16
NW = NC * NS

B = BATCH * POS
B_PER_W = B // NW
CH = 16
N_CHUNKS = B_PER_W // CH
N_BUF = 6
DEPTH = N_BUF // 2


def _embed_gather(tokens_flat, w_e):
    mesh = plsc.VectorSubcoreMesh(core_axis_name="c", subcore_axis_name="s")
    idx = tokens_flat.reshape(NW, N_CHUNKS, CH)

    scratch = [pltpu.VMEM((N_CHUNKS, CH), jnp.int32)]
    scratch += [pltpu.VMEM((CH, D_MODEL), jnp.float32) for _ in range(N_BUF)]
    scratch += [pltpu.SemaphoreType.DMA for _ in range(2 * N_BUF)]

    @functools.partial(
        pl.kernel,
        mesh=mesh,
        out_type=jax.ShapeDtypeStruct((B, D_MODEL), jnp.float32),
        scratch_types=scratch,
    )
    def k(table_hbm, idx_hbm, out_hbm, idx_v, *bufs_and_sems):
        rows = bufs_and_sems[:N_BUF]
        gsem = bufs_and_sems[N_BUF:2 * N_BUF]
        wsem = bufs_and_sems[2 * N_BUF:]
        wid = lax.axis_index("s") * NC + lax.axis_index("c")
        base = wid * B_PER_W
        pltpu.sync_copy(idx_hbm.at[wid], idx_v)

        def gather(c):
            b = c % N_BUF
            pltpu.make_async_copy(
                table_hbm.at[idx_v.at[c]], rows[b], gsem[b]).start()

        def write(c):
            b = c % N_BUF
            return pltpu.make_async_copy(
                rows[b], out_hbm.at[pl.ds(base + c * CH, CH)], wsem[b])

        for c in range(min(DEPTH, N_CHUNKS)):
            gather(c)
        for c in range(N_CHUNKS):
            b = c % N_BUF
            pltpu.make_async_copy(
                table_hbm.at[idx_v.at[c]], rows[b], gsem[b]).wait()
            write(c).start()
            n = c + DEPTH
            if n < N_CHUNKS:
                if n >= N_BUF:
                    write(n - N_BUF).wait()
                gather(n)
        for c in range(max(0, N_CHUNKS - N_BUF), N_CHUNKS):
            write(c).wait()

        return None

    return k(w_e, idx)


def kernel(tokens, W_E):
    tokens_flat = tokens.reshape(B).astype(jnp.int32)
    out = _embed_gather(tokens_flat, W_E)
    return out.reshape(BATCH, POS, D_MODEL)
